# Initial kernel scaffold; baseline (speedup 1.0000x reference)
import jax
import jax.numpy as jnp
from jax.experimental import pallas as pl


def kernel(input_tensor, indices, values):
    raise NotImplementedError("write your pallas kernel here")


# speedup vs baseline: 6.4382x; 6.4382x over previous
import functools

import jax
import jax.numpy as jnp
from jax import lax
from jax.experimental import pallas as pl
from jax.experimental.pallas import tpu as pltpu
from jax.experimental.pallas import tpu_sc as plsc

NC = 2
NS = 16
NW = NC * NS
LANES = 16


def _make_sc_partial(N, K, V, V_pad, CH):
    E = N * K
    assert E % NW == 0
    per_w = E // NW
    assert per_w % CH == 0 and CH % LANES == 0 and CH <= 128
    n_ch = per_w // CH
    k_shift = K.bit_length() - 1
    assert K == (1 << k_shift)

    mesh = plsc.VectorSubcoreMesh(core_axis_name="c", subcore_axis_name="s")

    @functools.partial(
        pl.kernel,
        out_type=jax.ShapeDtypeStruct((NW * V_pad,), jnp.float32),
        mesh=mesh,
        compiler_params=pltpu.CompilerParams(needs_layout_passes=False),
        scratch_types=[
            pltpu.VMEM((per_w,), jnp.int32),
            pltpu.VMEM((per_w,), jnp.float32),
            pltpu.VMEM((CH,), jnp.int32),
            pltpu.VMEM((CH,), jnp.float32),
            pltpu.VMEM((V_pad,), jnp.float32),
            pltpu.SemaphoreType.DMA,
        ],
    )
    def sc_partial(input_hbm, idx_hbm, val_hbm, out_hbm,
                   idx_v, val_v, off_v, gat_v, acc_v, sem):
        wid = lax.axis_index("c") * NS + lax.axis_index("s")
        base = wid * per_w

        pltpu.sync_copy(idx_hbm.at[pl.ds(base, per_w)], idx_v)
        pltpu.sync_copy(val_hbm.at[pl.ds(base, per_w)], val_v)

        zeros = jnp.zeros((LANES,), jnp.float32)

        def zero_body(i, _):
            acc_v[pl.ds(i * LANES, LANES)] = zeros
            return _

        lax.fori_loop(0, V_pad // LANES, zero_body, None)

        lanes = lax.iota(jnp.int32, 16)

        def chunk_body(c, _):
            e0 = c * CH
            for s in range(CH // LANES):
                ii = idx_v[pl.ds(e0 + s * LANES, LANES)]
                row = (base + e0 + s * LANES + lanes) >> k_shift
                off_v[pl.ds(s * LANES, LANES)] = row * V + ii
            pltpu.async_copy(input_hbm.at[off_v], gat_v, sem).wait()
            for s in range(CH // LANES):
                g = gat_v[pl.ds(s * LANES, LANES)]
                vv = val_v[pl.ds(e0 + s * LANES, LANES)]
                ii = idx_v[pl.ds(e0 + s * LANES, LANES)]
                plsc.addupdate_scatter(acc_v, [ii], g * vv)
            return _

        lax.fori_loop(0, n_ch, chunk_body, None)

        pltpu.sync_copy(acc_v, out_hbm.at[pl.ds(wid * V_pad, V_pad)])

    return sc_partial


def _make_tc_finish(V, V_pad):
    def body(p_ref, rs_ref, al_ref):
        p = p_ref[...]
        s = jnp.sum(p, axis=0, keepdims=True)
        col = lax.broadcasted_iota(jnp.int32, (1, V_pad), 1)
        valid = col < V
        rs_ref[...] = s
        m = jnp.max(jnp.where(valid, s, -jnp.inf))
        e = jnp.where(valid, jnp.exp(s - m), 0.0)
        al_ref[...] = e / jnp.sum(e)

    return pl.pallas_call(
        body,
        out_shape=(
            jax.ShapeDtypeStruct((1, V_pad), jnp.float32),
            jax.ShapeDtypeStruct((1, V_pad), jnp.float32),
        ),
    )


def kernel(input_tensor, indices, values):
    N, V = input_tensor.shape
    _, K = indices.shape
    V_pad = ((V + 127) // 128) * 128
    sc_partial = _make_sc_partial(N, K, V, V_pad, CH=80)
    tc_finish = _make_tc_finish(V, V_pad)

    partials = sc_partial(
        input_tensor.reshape(-1),
        indices.reshape(-1),
        values.reshape(-1),
    )
    rs, alpha = tc_finish(partials.reshape(NW, V_pad))
    return alpha[0, :V], rs[0, :V]


# speedup vs baseline: 18.5655x; 2.8836x over previous
import functools

import jax
import jax.numpy as jnp
from jax import lax
from jax.experimental import pallas as pl
from jax.experimental.pallas import tpu as pltpu
from jax.experimental.pallas import tpu_sc as plsc

NC = 2
NS = 16
NW = NC * NS
LANES = 16
G = 4
NBUF = 2


def _make_sc_partial(N, K, V, V_pad):
    assert N % G == 0 and K % LANES == 0
    n_groups = N // G
    max_trips = (n_groups + NW - 1) // NW

    mesh = plsc.VectorSubcoreMesh(core_axis_name="c", subcore_axis_name="s")

    @functools.partial(
        pl.kernel,
        out_type=jax.ShapeDtypeStruct((NW * V_pad,), jnp.float32),
        mesh=mesh,
        compiler_params=pltpu.CompilerParams(needs_layout_passes=False),
        scratch_types=[
            pltpu.VMEM((NBUF, G, V), jnp.float32),
            pltpu.VMEM((NBUF, G, K), jnp.int32),
            pltpu.VMEM((NBUF, G, K), jnp.float32),
            pltpu.VMEM((V_pad,), jnp.float32),
            pltpu.SemaphoreType.DMA((NBUF,)),
        ],
    )
    def sc_partial(input_hbm, idx_hbm, val_hbm, out_hbm,
                   rows_v, idx_v, val_v, acc_v, sems):
        wid = lax.axis_index("c") * NS + lax.axis_index("s")

        zeros = jnp.zeros((LANES,), jnp.float32)

        def zero_body(i, _):
            acc_v[pl.ds(i * LANES, LANES)] = zeros
            return _

        lax.fori_loop(0, V_pad // LANES, zero_body, None)

        def copies(g, b):
            r = g * G
            return (
                pltpu.make_async_copy(input_hbm.at[pl.ds(r, G), :], rows_v.at[b], sems.at[b]),
                pltpu.make_async_copy(idx_hbm.at[pl.ds(r, G), :], idx_v.at[b], sems.at[b]),
                pltpu.make_async_copy(val_hbm.at[pl.ds(r, G), :], val_v.at[b], sems.at[b]),
            )

        def issue(g, b):
            @pl.when(g < n_groups)
            def _():
                for cp in copies(g, b):
                    cp.start()

        def process(g, b):
            @pl.when(g < n_groups)
            def _():
                for cp in copies(g, b):
                    cp.wait()
                for s in range(G * K // LANES):
                    rr = s // (K // LANES)
                    q = s % (K // LANES)
                    ii = idx_v[b, rr, pl.ds(q * LANES, LANES)]
                    vv = val_v[b, rr, pl.ds(q * LANES, LANES)]
                    rvec = jnp.full((LANES,), rr, jnp.int32)
                    gg = plsc.load_gather(rows_v.at[b], [rvec, ii])
                    plsc.addupdate_scatter(acc_v, [ii], gg * vv)

        for b in range(NBUF):
            issue(wid + b * NW, b)

        def trip(t, _):
            for b in range(NBUF):
                g = wid + (t * NBUF + b) * NW
                process(g, b)
                issue(g + NBUF * NW, b)
            return _

        lax.fori_loop(0, (max_trips + NBUF - 1) // NBUF, trip, None)

        pltpu.sync_copy(acc_v, out_hbm.at[pl.ds(wid * V_pad, V_pad)])

    return sc_partial


def _make_tc_finish(V, V_pad):
    def body(p_ref, rs_ref, al_ref):
        p = p_ref[...]
        s = jnp.sum(p, axis=0, keepdims=True)
        col = lax.broadcasted_iota(jnp.int32, (1, V_pad), 1)
        valid = col < V
        rs_ref[...] = s
        m = jnp.max(jnp.where(valid, s, -jnp.inf))
        e = jnp.where(valid, jnp.exp(s - m), 0.0)
        al_ref[...] = e / jnp.sum(e)

    return pl.pallas_call(
        body,
        out_shape=(
            jax.ShapeDtypeStruct((1, V_pad), jnp.float32),
            jax.ShapeDtypeStruct((1, V_pad), jnp.float32),
        ),
    )


def kernel(input_tensor, indices, values):
    N, V = input_tensor.shape
    _, K = indices.shape
    V_pad = ((V + 127) // 128) * 128
    sc_partial = _make_sc_partial(N, K, V, V_pad)
    tc_finish = _make_tc_finish(V, V_pad)

    partials = sc_partial(input_tensor, indices, values)
    rs, alpha = tc_finish(partials.reshape(NW, V_pad))
    return alpha[0, :V], rs[0, :V]
